# baseline (speedup 1.0000x reference)
import functools

import jax
import jax.numpy as jnp
from jax import lax
from jax.experimental import pallas as pl
from jax.experimental.pallas import tpu as pltpu
from jax.experimental.pallas import tpu_sc as plsc

ROWS, COLS = 16384, 20
D = 64
B = ROWS * COLS
NC, NS = 2, 16
NW = NC * NS
BPW = B // NW
C = 128
NCH = BPW // C

_MESH = plsc.VectorSubcoreMesh(core_axis_name="c", subcore_axis_name="s")


@functools.partial(
    pl.kernel,
    mesh=_MESH,
    compiler_params=pltpu.CompilerParams(use_tc_tiling_on_sc=False),
    out_type=jax.ShapeDtypeStruct((B, D), jnp.float32),
    scratch_types=[
        pltpu.VMEM((NCH, C), jnp.int32),
        pltpu.VMEM((C, D), jnp.float32),
        pltpu.SemaphoreType.DMA,
    ],
)
def _sc_gather(idx_hbm, table_hbm, out_hbm, idx_v, rows_v, sem):
    wid = lax.axis_index("s") * NC + lax.axis_index("c")
    base = wid * BPW
    pltpu.sync_copy(idx_hbm.at[wid], idx_v)

    def body(j, carry):
        pltpu.async_copy(table_hbm.at[idx_v.at[j]], rows_v, sem).wait()
        pltpu.sync_copy(rows_v, out_hbm.at[pl.ds(base + j * C, C)])
        return carry

    lax.fori_loop(0, NCH, body, 0, unroll=False)


def kernel(inp, table):
    idx = inp.reshape(NW, NCH, C).astype(jnp.int32)
    out = _sc_gather(idx, table)
    return out.reshape(ROWS, COLS, D)


# speedup vs baseline: 1.0627x; 1.0627x over previous
import functools

import jax
import jax.numpy as jnp
from jax import lax
from jax.experimental import pallas as pl
from jax.experimental.pallas import tpu as pltpu
from jax.experimental.pallas import tpu_sc as plsc

ROWS, COLS = 16384, 20
D = 64
B = ROWS * COLS
NC, NS = 2, 16
NW = NC * NS
BPW = B // NW
C = 128
NCH = BPW // C
NBUF = 4
NITER = NCH // NBUF

_MESH = plsc.VectorSubcoreMesh(core_axis_name="c", subcore_axis_name="s")


@functools.partial(
    pl.kernel,
    mesh=_MESH,
    compiler_params=pltpu.CompilerParams(use_tc_tiling_on_sc=False),
    out_type=jax.ShapeDtypeStruct((B, D), jnp.float32),
    scratch_types=[
        pltpu.VMEM((NCH, C), jnp.int32),
        pltpu.VMEM((NBUF, C, D), jnp.float32),
        [pltpu.SemaphoreType.DMA] * NBUF,
        [pltpu.SemaphoreType.DMA] * NBUF,
    ],
)
def _sc_gather(idx_hbm, table_hbm, out_hbm, idx_v, rows_v, gsem, osem):
    wid = lax.axis_index("s") * NC + lax.axis_index("c")
    base = wid * BPW
    pltpu.sync_copy(idx_hbm.at[wid], idx_v)

    def gather(j, b):
        pltpu.async_copy(table_hbm.at[idx_v.at[j]], rows_v.at[b], gsem[b])

    def wait_gather(b):
        pltpu.make_async_copy(table_hbm.at[idx_v.at[0]], rows_v.at[b], gsem[b]).wait()

    def writeback(j, b):
        pltpu.async_copy(rows_v.at[b], out_hbm.at[pl.ds(base + j * C, C)], osem[b])

    def wait_writeback(b):
        pltpu.make_async_copy(
            rows_v.at[b], out_hbm.at[pl.ds(base, C)], osem[b]).wait()

    for b in range(NBUF):
        gather(b, b)

    def outer(i, carry):
        j0 = i * NBUF
        for b in range(NBUF):
            wait_gather(b)
            writeback(j0 + b, b)
        for b in range(NBUF):
            wait_writeback(b)
            gather(j0 + NBUF + b, b)
        return carry

    lax.fori_loop(0, NITER - 1, outer, 0, unroll=False)

    j0 = (NITER - 1) * NBUF
    for b in range(NBUF):
        wait_gather(b)
        writeback(j0 + b, b)
    for b in range(NBUF):
        wait_writeback(b)


def kernel(inp, table):
    idx = inp.reshape(NW, NCH, C).astype(jnp.int32)
    out = _sc_gather(idx, table)
    return out.reshape(ROWS, COLS, D)
